# Initial kernel scaffold; baseline (speedup 1.0000x reference)
import jax
import jax.numpy as jnp
from jax.experimental import pallas as pl


def kernel(x, batch, pe):
    raise NotImplementedError("write your pallas kernel here")


# speedup vs baseline: 8.0293x; 8.0293x over previous
import functools

import jax
import jax.numpy as jnp
from jax import lax
from jax.experimental import pallas as pl
from jax.experimental.pallas import tpu as pltpu
from jax.experimental.pallas import tpu_sc as plsc

D_MODEL = 256
MAX_LEN = 4096
TOTAL_TOK = 32768

NUM_CORES = 2
NUM_SUBCORES = 16
LANES = 16
NUM_WORKERS = NUM_CORES * NUM_SUBCORES
CHUNK = TOTAL_TOK // NUM_WORKERS
NVREG = CHUNK // LANES
BLK = 128
NBLK = CHUNK // BLK

_mesh = plsc.VectorSubcoreMesh(
    core_axis_name="c", subcore_axis_name="s",
    num_cores=NUM_CORES, num_subcores=NUM_SUBCORES,
)


@functools.partial(
    pl.kernel,
    out_type=jax.ShapeDtypeStruct((TOTAL_TOK, D_MODEL), jnp.float32),
    mesh=_mesh,
    compiler_params=pltpu.CompilerParams(needs_layout_passes=False),
    scratch_types=[
        pltpu.VMEM((TOTAL_TOK,), jnp.int32),
        pltpu.VMEM((NBLK, BLK), jnp.int32),
        pltpu.VMEM((BLK, D_MODEL), jnp.float32),
        pltpu.VMEM((BLK, D_MODEL), jnp.float32),
        pltpu.SemaphoreType.DMA,
    ],
)
def _pe_add(x_hbm, batch_hbm, pe_hbm, out_hbm, bvals, idx, buf, pbuf, sem):
    wid = lax.axis_index("s") * NUM_CORES + lax.axis_index("c")
    base = wid * CHUNK
    iot = lax.iota(jnp.int32, LANES)

    pltpu.sync_copy(batch_hbm, bvals)

    firstvec = bvals[pl.ds(base, LANES)]
    firstval = jnp.max(jnp.where(iot == 0, firstvec, jnp.int32(-1)))

    def count_body(j, acc):
        v = bvals[pl.ds(j * LANES, LANES)]
        return acc + (v < firstval).astype(jnp.int32)

    acc = lax.fori_loop(0, wid * NVREG, count_body,
                        jnp.zeros((LANES,), jnp.int32))
    seg_start0 = jnp.sum(acc)

    run_carry = seg_start0
    for j in range(NVREG):
        g0 = base + j * LANES
        gi = g0 + iot
        cur = bvals[pl.ds(g0, LANES)]
        prev = plsc.load_gather(bvals, [jnp.maximum(gi - 1, 0)])
        bnd = cur != jnp.where(gi == 0, jnp.int32(-1), prev)
        cand = jnp.where(bnd, gi, jnp.int32(-1))
        run = jnp.maximum(plsc.cummax(cand), run_carry)
        pos = jnp.minimum(gi - run, jnp.int32(MAX_LEN - 1))
        idx[j // (BLK // LANES), pl.ds((j % (BLK // LANES)) * LANES, LANES)] = pos
        run_carry = jnp.max(run)

    for t in range(NBLK):
        r0 = base + t * BLK
        pltpu.sync_copy(x_hbm.at[pl.ds(r0, BLK)], buf)
        pltpu.async_copy(pe_hbm.at[idx.at[t]], pbuf, sem).wait()

        def add_row(r, _, t=t):
            for c in range(D_MODEL // LANES):
                v = buf[r, pl.ds(c * LANES, LANES)]
                plsc.addupdate(pbuf.at[r, pl.ds(c * LANES, LANES)], v)
            return 0

        lax.fori_loop(0, BLK, add_row, 0)
        pltpu.sync_copy(pbuf, out_hbm.at[pl.ds(r0, BLK)])


def kernel(x, batch, pe):
    return _pe_add(x, batch.astype(jnp.int32), pe)
